# speedup vs baseline: 1.3705x; 1.3705x over previous
import functools

import jax
import jax.numpy as jnp
from jax import lax
from jax.experimental import pallas as pl
from jax.experimental.pallas import tpu as pltpu
from jax.experimental.pallas import tpu_sc as plsc

_N = 10000
_K = 32
_D = 128

_NW = 32
_NPW = 320
_NPAD = _NW * _NPW
_CB = 4
_IDXC = _CB * _K
_CPW = _NPW // _CB
_LANES = 16
_NV = _D // _LANES


def _sc_neighbor_sum(x, edge_groups):
    mesh = plsc.VectorSubcoreMesh(core_axis_name="c", subcore_axis_name="s")

    @functools.partial(
        pl.kernel,
        out_type=jax.ShapeDtypeStruct((_NPAD, _D), jnp.float32),
        mesh=mesh,
        scratch_types=[
            pltpu.VMEM((_CPW, _IDXC), jnp.int32),
            pltpu.VMEM((_IDXC, _D), jnp.float32),
            pltpu.VMEM((_IDXC, _D), jnp.float32),
            pltpu.VMEM((_NPW, _D), jnp.float32),
            pltpu.SemaphoreType.DMA,
            pltpu.SemaphoreType.DMA,
        ],
    )
    def body(x_hbm, edge_hbm, out_hbm, idx_v, rows0, rows1, out_v, sem0, sem1):
        wid = lax.axis_index("s") * 2 + lax.axis_index("c")

        pltpu.sync_copy(edge_hbm.at[wid], idx_v)

        def start(g, rows_ref, sem):
            pltpu.async_copy(x_hbm.at[idx_v.at[g]], rows_ref, sem)

        def wait(rows_ref, sem):
            pltpu.make_async_copy(x_hbm.at[pl.ds(0, _IDXC)], rows_ref, sem).wait()

        def compute(g, rows_ref):
            base_slot = g * _CB
            for b in range(_CB):
                def kstep(k, accs, _b=b):
                    r = _b * _K + 2 * k
                    accs = tuple(
                        accs[c] + rows_ref[r, pl.ds(c * _LANES, _LANES)]
                        for c in range(_NV)
                    )
                    return tuple(
                        accs[c] + rows_ref[r + 1, pl.ds(c * _LANES, _LANES)]
                        for c in range(_NV)
                    )

                zeros = tuple(jnp.zeros((_LANES,), jnp.float32) for _ in range(_NV))
                accs = lax.fori_loop(0, _K // 2, kstep, zeros)
                for c in range(_NV):
                    out_v[base_slot + b, pl.ds(c * _LANES, _LANES)] = accs[c]

        start(0, rows0, sem0)
        start(1, rows1, sem1)

        def pair_body(p, carry):
            g = 2 * p
            wait(rows0, sem0)
            compute(g, rows0)

            @pl.when(p + 1 < _CPW // 2)
            def _():
                start(g + 2, rows0, sem0)

            wait(rows1, sem1)
            compute(g + 1, rows1)

            @pl.when(p + 1 < _CPW // 2)
            def _():
                start(g + 3, rows1, sem1)

            return carry

        lax.fori_loop(0, _CPW // 2, pair_body, 0)

        pltpu.sync_copy(out_v, out_hbm.at[pl.ds(wid * _NPW, _NPW)])

    return body(x, edge_groups)


def _tc_body(x_ref, ns_ref, eps_ref, wt_ref, b_ref, o_ref):
    h = (1.0 + eps_ref[0, 0]) * x_ref[...] + ns_ref[...]
    o_ref[...] = (
        jnp.dot(h, wt_ref[...], preferred_element_type=jnp.float32) + b_ref[...]
    )


def _tc_linear(x, nsum_pad, eps11, wt, b1):
    br = 1000
    return pl.pallas_call(
        _tc_body,
        grid=(_N // br,),
        in_specs=[
            pl.BlockSpec((br, _D), lambda i: (i, 0)),
            pl.BlockSpec((br, _D), lambda i: (i, 0)),
            pl.BlockSpec(memory_space=pltpu.SMEM),
            pl.BlockSpec((_D, _D), lambda i: (0, 0)),
            pl.BlockSpec((1, _D), lambda i: (0, 0)),
        ],
        out_specs=pl.BlockSpec((br, _D), lambda i: (i, 0)),
        out_shape=jax.ShapeDtypeStruct((_N, _D), jnp.float32),
    )(x, nsum_pad, eps11, wt, b1)


def kernel(x, edge_index, eps, W, b):
    pad = jnp.zeros(((_NPAD - _N) * _K,), jnp.int32)
    edge_groups = jnp.concatenate([edge_index.reshape(-1), pad]).reshape(
        _NW, _CPW, _IDXC
    )
    nsum = _sc_neighbor_sum(x, edge_groups)
    return _tc_linear(x, nsum, eps.reshape(1, 1), W.T, b.reshape(1, _D))
